# baseline (speedup 1.0000x reference)
import functools

import numpy as np
import jax
from jax import lax
import jax.numpy as jnp
from jax.experimental import pallas as pl
from jax.experimental.pallas import tpu as pltpu
from jax.experimental.pallas import tpu_sc as plsc


def _line_constants(num_rows: int):
    rng = np.random.default_rng(0)
    rand_row = int(rng.integers(0, num_rows))
    x_min, alpha = 1.0728769e-07, 1.0868737
    r = float(rng.random())
    relative_error = x_min * (1.0 - r) ** (-1.0 / (alpha - 1.0))
    return rand_row, relative_error


_CHUNK_ROWS = 64


def _sc_clone(forward_input, n_rows, n_cols):
    n_workers = 32
    rows_per = n_rows // n_workers
    n_chunks = rows_per // _CHUNK_ROWS

    mesh = plsc.VectorSubcoreMesh(core_axis_name="c", subcore_axis_name="s")

    @functools.partial(
        pl.kernel,
        out_type=jax.ShapeDtypeStruct((n_rows, n_cols), forward_input.dtype),
        mesh=mesh,
        scratch_types=[
            pltpu.VMEM((_CHUNK_ROWS, n_cols), forward_input.dtype),
        ],
    )
    def sc_kernel(x_hbm, o_hbm, buf):
        wid = lax.axis_index("s") * 2 + lax.axis_index("c")
        base = wid * rows_per
        for i in range(n_chunks):
            sl = pl.ds(base + i * _CHUNK_ROWS, _CHUNK_ROWS)
            pltpu.sync_copy(x_hbm.at[sl], buf)
            pltpu.sync_copy(buf, o_hbm.at[sl])

    return sc_kernel(forward_input)


def _tc_fix_row(cloned, rand_row, rel_err, n_cols):
    grp = rand_row - (rand_row % 8)
    off = rand_row - grp

    def body(ref_in, ref_out):
        ref_out[...] = ref_in[...]
        ref_out[off, :] = ref_in[off, :] * jnp.float32(rel_err)

    return pl.pallas_call(
        body,
        grid=(1,),
        in_specs=[pl.BlockSpec((8, n_cols), lambda i: (grp // 8, 0))],
        out_specs=pl.BlockSpec((8, n_cols), lambda i: (grp // 8, 0)),
        out_shape=jax.ShapeDtypeStruct(cloned.shape, cloned.dtype),
        input_output_aliases={0: 0},
    )(cloned)


def kernel(forward_input):
    n_rows, n_cols = forward_input.shape
    rand_row, rel_err = _line_constants(n_rows)
    cloned = _sc_clone(forward_input, n_rows, n_cols)
    return _tc_fix_row(cloned, rand_row, rel_err, n_cols)


# speedup vs baseline: 1.0374x; 1.0374x over previous
import functools

import numpy as np
import jax
from jax import lax
import jax.numpy as jnp
from jax.experimental import pallas as pl
from jax.experimental.pallas import tpu as pltpu
from jax.experimental.pallas import tpu_sc as plsc


def _line_constants(num_rows: int):
    rng = np.random.default_rng(0)
    rand_row = int(rng.integers(0, num_rows))
    x_min, alpha = 1.0728769e-07, 1.0868737
    r = float(rng.random())
    relative_error = x_min * (1.0 - r) ** (-1.0 / (alpha - 1.0))
    return rand_row, relative_error


_CHUNK_ROWS = 16
_NBUF = 6
_AHEAD = 3


def _sc_clone(forward_input, n_rows, n_cols):
    n_workers = 32
    rows_per = n_rows // n_workers
    n_chunks = rows_per // _CHUNK_ROWS

    mesh = plsc.VectorSubcoreMesh(core_axis_name="c", subcore_axis_name="s")

    @functools.partial(
        pl.kernel,
        out_type=jax.ShapeDtypeStruct((n_rows, n_cols), forward_input.dtype),
        mesh=mesh,
        scratch_types=[
            pltpu.VMEM((_NBUF, _CHUNK_ROWS, n_cols), forward_input.dtype),
            pltpu.SemaphoreType.DMA((_NBUF,)),
            pltpu.SemaphoreType.DMA((_NBUF,)),
        ],
    )
    def sc_kernel(x_hbm, o_hbm, bufs, rsems, wsems):
        wid = lax.axis_index("s") * 2 + lax.axis_index("c")
        base = wid * rows_per

        def read(i):
            return pltpu.make_async_copy(
                x_hbm.at[pl.ds(base + i * _CHUNK_ROWS, _CHUNK_ROWS)],
                bufs.at[i % _NBUF],
                rsems.at[i % _NBUF],
            )

        def write(i):
            return pltpu.make_async_copy(
                bufs.at[i % _NBUF],
                o_hbm.at[pl.ds(base + i * _CHUNK_ROWS, _CHUNK_ROWS)],
                wsems.at[i % _NBUF],
            )

        reads = [None] * n_chunks
        writes = [None] * n_chunks
        for i in range(min(_AHEAD, n_chunks)):
            reads[i] = read(i)
            reads[i].start()
        for i in range(n_chunks):
            r = i + _AHEAD
            if r < n_chunks:
                if r >= _NBUF:
                    writes[r - _NBUF].wait()
                reads[r] = read(r)
                reads[r].start()
            reads[i].wait()
            writes[i] = write(i)
            writes[i].start()
        for i in range(max(n_chunks - _NBUF, 0), n_chunks):
            writes[i].wait()

    return sc_kernel(forward_input)


def _tc_fix_row(cloned, rand_row, rel_err, n_cols):
    grp = rand_row - (rand_row % 8)
    off = rand_row - grp

    def body(ref_in, ref_out):
        ref_out[...] = ref_in[...]
        ref_out[off, :] = ref_in[off, :] * jnp.float32(rel_err)

    return pl.pallas_call(
        body,
        grid=(1,),
        in_specs=[pl.BlockSpec((8, n_cols), lambda i: (grp // 8, 0))],
        out_specs=pl.BlockSpec((8, n_cols), lambda i: (grp // 8, 0)),
        out_shape=jax.ShapeDtypeStruct(cloned.shape, cloned.dtype),
        input_output_aliases={0: 0},
    )(cloned)


def kernel(forward_input):
    n_rows, n_cols = forward_input.shape
    rand_row, rel_err = _line_constants(n_rows)
    cloned = _sc_clone(forward_input, n_rows, n_cols)
    return _tc_fix_row(cloned, rand_row, rel_err, n_cols)


# speedup vs baseline: 1.6836x; 1.6228x over previous
import numpy as np
import jax
import jax.numpy as jnp
from jax.experimental import pallas as pl


def _line_constants(num_rows: int):
    rng = np.random.default_rng(0)
    rand_row = int(rng.integers(0, num_rows))
    x_min, alpha = 1.0728769e-07, 1.0868737
    r = float(rng.random())
    relative_error = x_min * (1.0 - r) ** (-1.0 / (alpha - 1.0))
    return rand_row, relative_error


_BLOCK_ROWS = 2048


def kernel(forward_input):
    n_rows, n_cols = forward_input.shape
    rand_row, rel_err = _line_constants(n_rows)

    block_rows = _BLOCK_ROWS
    grid = n_rows // block_rows
    target_block = rand_row // block_rows
    row_off = rand_row % block_rows

    def body(x_ref, o_ref):
        i = pl.program_id(0)
        o_ref[...] = x_ref[...]

        @pl.when(i == target_block)
        def _():
            o_ref[row_off, :] = x_ref[row_off, :] * jnp.float32(rel_err)

    return pl.pallas_call(
        body,
        grid=(grid,),
        in_specs=[pl.BlockSpec((block_rows, n_cols), lambda i: (i, 0))],
        out_specs=pl.BlockSpec((block_rows, n_cols), lambda i: (i, 0)),
        out_shape=jax.ShapeDtypeStruct((n_rows, n_cols), forward_input.dtype),
    )(forward_input)
